# baseline (speedup 1.0000x reference)
import functools

import jax
import jax.numpy as jnp
from jax import lax
from jax.experimental import pallas as pl
from jax.experimental.pallas import tpu as pltpu
from jax.experimental.pallas import tpu_sc as plsc

_R0 = 0.001
_QB = 256
_RB = 512


def _fps_body(xl_ref, xyz_ref, *, npoint, n, g):
    r = n // 128
    x = xl_ref[:, 0]
    y = xl_ref[:, 1]
    z = xl_ref[:, 2]
    lin = (lax.broadcasted_iota(jnp.int32, (g, r, 128), 1) * 128
           + lax.broadcasted_iota(jnp.int32, (g, r, 128), 2))
    d0 = jnp.full((g, r, 128), 1e10, dtype=jnp.float32)

    def body(i, carry):
        f, d = carry
        msk = lin == f
        zero = jnp.zeros((g, r, 128), jnp.float32)
        cx = jnp.sum(jnp.where(msk, x, zero), axis=(1, 2), keepdims=True)
        cy = jnp.sum(jnp.where(msk, y, zero), axis=(1, 2), keepdims=True)
        cz = jnp.sum(jnp.where(msk, z, zero), axis=(1, 2), keepdims=True)
        for c in range(g):
            cc = jnp.concatenate(
                [cx[c, :, 0:1], cy[c, :, 0:1], cz[c, :, 0:1]], axis=1)
            xyz_ref[c, pl.ds(i, 1), :] = cc
        dx = x - cx
        dy = y - cy
        dz = z - cz
        dd = dx * dx + dy * dy + dz * dz
        d = jnp.minimum(d, dd)
        m = jnp.max(d, axis=(1, 2), keepdims=True)
        f2 = jnp.min(jnp.where(d == m, lin, n), axis=(1, 2), keepdims=True)
        return f2.astype(jnp.int32), d

    lax.fori_loop(0, npoint, body,
                  (jnp.zeros((g, 1, 1), jnp.int32), d0))


def _fps(xyz_rows, npoint):
    g, n, _ = xyz_rows.shape
    r = n // 128
    xl = xyz_rows.transpose(0, 2, 1).reshape(g, 3, r, 128)
    body = functools.partial(_fps_body, npoint=npoint, n=n, g=g)
    sxyz = pl.pallas_call(
        body,
        out_shape=jax.ShapeDtypeStruct((g, npoint, 3), jnp.float32),
    )(xl)
    return sxyz


def _neighbor_body(q_ref, s_ref, o_ref, *, nsample, v, mode, r2):
    q = q_ref[0]
    s = s_ref[0]
    dg = lax.dot_general(q, s, (((1,), (0,)), ((), ())),
                         preferred_element_type=jnp.float32)
    qn = jnp.sum(q * q, axis=1, keepdims=True)
    sn = jnp.sum(s * s, axis=0, keepdims=True)
    d = -2.0 * dg
    d = d + qn
    d = d + sn
    qb = q.shape[0]
    lane = lax.broadcasted_iota(jnp.int32, (qb, v), 1)
    bidx = pl.program_id(0)
    if mode == "ball":
        g = jnp.where(d > r2, v, lane)
        cols = []
        for _ in range(nsample):
            m = jnp.min(g, axis=1, keepdims=True)
            cols.append(m)
            g = jnp.where(g == m, v, g)
        first = cols[0]
        cols = [jnp.where(c == v, first, c) for c in cols]
        res = jnp.minimum(jnp.concatenate(cols, axis=1), v - 1)
    else:
        cols = []
        for _ in range(nsample):
            m = jnp.min(d, axis=1, keepdims=True)
            j = jnp.min(jnp.where(d == m, lane, v), axis=1, keepdims=True)
            cols.append(j)
            d = jnp.where(lane == j, jnp.inf, d)
        res = jnp.concatenate(cols, axis=1)
    o_ref[0] = res + bidx * v


def _neighbor(queries, sources, nsample, mode, r2=0.0):
    b, m, _ = queries.shape
    v = sources.shape[1]
    qb = min(_QB, m)
    sl = sources.transpose(0, 2, 1)
    body = functools.partial(_neighbor_body, nsample=nsample, v=v,
                             mode=mode, r2=r2)
    idx = pl.pallas_call(
        body,
        grid=(b, m // qb),
        in_specs=[
            pl.BlockSpec((1, qb, 3), lambda i, j: (i, j, 0)),
            pl.BlockSpec((1, 3, v), lambda i, j: (i, 0, 0)),
        ],
        out_specs=pl.BlockSpec((1, qb, nsample), lambda i, j: (i, j, 0)),
        out_shape=jax.ShapeDtypeStruct((b, m, nsample), jnp.int32),
    )(queries, sl)
    return idx


def _sc_gather(table, idx, chunk):
    v, dd = table.shape
    (btot,) = idx.shape
    nc, ns = 2, 16
    nw = nc * ns
    b_per_w = btot // nw
    nchunks = b_per_w // chunk
    mesh = plsc.VectorSubcoreMesh(core_axis_name="c", subcore_axis_name="s")

    @functools.partial(
        pl.kernel,
        mesh=mesh,
        compiler_params=pltpu.CompilerParams(use_tc_tiling_on_sc=False),
        out_type=jax.ShapeDtypeStruct((btot, dd), jnp.float32),
        scratch_types=[
            pltpu.VMEM((chunk,), jnp.int32),
            pltpu.VMEM((chunk, dd), jnp.float32),
            pltpu.SemaphoreType.DMA,
        ],
    )
    def k(table_hbm, idx_hbm, out_hbm, idx_v, rows_v, sem):
        wid = lax.axis_index("s") * nc + lax.axis_index("c")
        base = wid * b_per_w
        for ci in range(nchunks):
            off = base + ci * chunk
            pltpu.sync_copy(idx_hbm.at[pl.ds(off, chunk)], idx_v)
            pltpu.async_copy(table_hbm.at[idx_v], rows_v, sem).wait()
            pltpu.sync_copy(rows_v, out_hbm.at[pl.ds(off, chunk)])

    return k(table, idx)


def _acc_stats(st_ref, y):
    s1 = jnp.sum(y, axis=0, keepdims=True)
    s2 = jnp.sum(y * y, axis=0, keepdims=True)

    @pl.when(pl.program_id(0) == 0)
    def _():
        st_ref[...] = jnp.zeros_like(st_ref)

    st_ref[...] += jnp.concatenate([s1, s2], axis=0)


def _bn_relu(h, stats, gamma, beta, cnt, relu=True):
    shp = (1,) * (h.ndim - 1) + (h.shape[-1],)
    mean = (stats[0:1, :] / cnt).reshape(shp)
    ex2 = (stats[1:2, :] / cnt).reshape(shp)
    var = ex2 - mean * mean
    y = gamma.reshape(shp) * (h - mean) / jnp.sqrt(var + 1e-5) + beta.reshape(shp)
    return jnp.maximum(y, 0.0) if relu else y


def _rowsT(h, w):
    return lax.dot_general(h, w, (((1,), (1,)), ((), ())),
                           preferred_element_type=jnp.float32)


def _grouped_lin_body(g_ref, nx_ref, w_ref, o_ref, st_ref):
    g = g_ref[...]
    nx = nx_ref[...]
    rb, k, dd = g.shape
    h = (g - nx[:, None, :]).reshape(rb * k, dd)
    y = _rowsT(h, w_ref[...])
    o_ref[...] = y.reshape(rb, k, y.shape[1])
    _acc_stats(st_ref, y)


def _norm_lin_body(h_ref, sti_ref, gam_ref, bet_ref, w_ref, o_ref, st_ref, *, cnt):
    h = h_ref[...]
    rb, k, cin = h.shape
    x = _bn_relu(h, sti_ref[...], gam_ref[...], bet_ref[...], cnt)
    y = _rowsT(x.reshape(rb * k, cin), w_ref[...])
    o_ref[...] = y.reshape(rb, k, y.shape[1])
    _acc_stats(st_ref, y)


def _norm_max_body(h_ref, sti_ref, gam_ref, bet_ref, o_ref, *, cnt):
    x = _bn_relu(h_ref[...], sti_ref[...], gam_ref[...], bet_ref[...], cnt)
    o_ref[...] = jnp.max(x, axis=1)


def _su_b_body(h_ref, sti_ref, gam_ref, bet_ref, f1_ref, w_ref, o_ref, st_ref, *, cnt):
    x = _bn_relu(h_ref[...], sti_ref[...], gam_ref[...], bet_ref[...], cnt)
    xm = jnp.max(x, axis=1)
    hc = jnp.concatenate([xm, f1_ref[...]], axis=1)
    y = _rowsT(hc, w_ref[...])
    o_ref[...] = y
    _acc_stats(st_ref, y)


def _su_c_body(h_ref, sti_ref, gam_ref, bet_ref, xyz_ref, o_ref, *, cnt):
    x = _bn_relu(h_ref[...], sti_ref[...], gam_ref[...], bet_ref[...], cnt)
    rb = x.shape[0]
    o_ref[...] = jnp.concatenate(
        [x, xyz_ref[...], jnp.zeros((rb, 13), jnp.float32)], axis=1)


def _fp_a_body(g_ref, pc_ref, fea_ref, w_ref, o_ref, st_ref):
    g = g_ref[...]
    p = pc_ref[...]
    f = fea_ref[...]
    px, py, pz = p[:, 0:1], p[:, 1:2], p[:, 2:3]
    ds = []
    for j in range(3):
        dx = g[:, j, 32:33] - px
        dy = g[:, j, 33:34] - py
        dz = g[:, j, 34:35] - pz
        ds.append(dx * dx + dy * dy + dz * dz)
    rc = [1.0 / (dj + 1e-8) for dj in ds]
    norm = rc[0] + rc[1] + rc[2]
    ws = [r / norm for r in rc]
    interp = (g[:, 0, 0:32] * ws[0] + g[:, 1, 0:32] * ws[1]
              + g[:, 2, 0:32] * ws[2])
    h = jnp.concatenate([interp, f], axis=1)
    y = _rowsT(h, w_ref[...])
    o_ref[...] = y
    _acc_stats(st_ref, y)


def _norm_relu_stats_body(h_ref, sti_ref, gam_ref, bet_ref, o_ref, st_ref, *, cnt):
    x = _bn_relu(h_ref[...], sti_ref[...], gam_ref[...], bet_ref[...], cnt)
    o_ref[...] = x
    _acc_stats(st_ref, x)


def _final_body(h_ref, sti_ref, gam_ref, bet_ref, w_ref, cb_ref, o_ref, *, cnt):
    x = _bn_relu(h_ref[...], sti_ref[...], gam_ref[...], bet_ref[...], cnt)
    o_ref[...] = _rowsT(x, w_ref[...]) + cb_ref[...]


def _full(shape):
    nd = len(shape)

    def im(i):
        return (0,) * nd

    return pl.BlockSpec(shape, im)


def _rowspec(blk_shape):
    nd = len(blk_shape)

    def im(i):
        return (i,) + (0,) * (nd - 1)

    return pl.BlockSpec(blk_shape, im)


_STATS = object()


def _call(body, outs, ins, rb):
    nrows = ins[0][0].shape[0]
    grid = (nrows // rb,)
    in_specs, args = [], []
    for a, kind in ins:
        if kind == "row":
            in_specs.append(_rowspec((rb,) + a.shape[1:]))
        else:
            in_specs.append(_full(a.shape))
        args.append(a)
    res_specs = []
    res_shapes = []
    for o in outs:
        if isinstance(o, tuple) and o[0] is _STATS:
            c = o[1]
            res_specs.append(_full((2, c)))
            res_shapes.append(jax.ShapeDtypeStruct((2, c), jnp.float32))
        else:
            res_specs.append(_rowspec((rb,) + o.shape[1:]))
            res_shapes.append(o)
    return pl.pallas_call(
        body,
        grid=grid,
        in_specs=in_specs,
        out_specs=res_specs if len(res_specs) > 1 else res_specs[0],
        out_shape=res_shapes if len(res_shapes) > 1 else res_shapes[0],
    )(*args)


def _sds(shape):
    return jax.ShapeDtypeStruct(shape, jnp.float32)


def _process(pc, fea, params, l1_xyz, l2_xyz):
    b, n, _ = pc.shape
    m1, m2 = n // 2, n // 8
    f32 = jnp.float32

    ball1 = _neighbor(l1_xyz, pc, 16, "ball", (_R0 * 8) ** 2)
    ball2 = _neighbor(l2_xyz, l1_xyz, 16, "ball", (_R0 * 16) ** 2)
    su = _neighbor(l1_xyz, l2_xyz, 8, "ball", (_R0 * 12) ** 2)
    fp = _neighbor(pc, l1_xyz, 3, "knn")

    (w1, g1, be1), (w2, g2, be2) = params['sa1']
    t0 = jnp.concatenate([pc, fea, jnp.zeros((b, n, 3), f32)], -1)
    gr1 = _sc_gather(t0.reshape(b * n, 16), ball1.reshape(-1), 4096)
    gr1 = gr1.reshape(b * m1, 16, 16)
    nx1 = jnp.concatenate([l1_xyz, jnp.zeros((b, m1, 13), f32)], -1)
    nx1 = nx1.reshape(b * m1, 16)
    w1p = jnp.pad(w1, ((0, 0), (0, 3)))
    h1, st1 = _call(_grouped_lin_body,
                    [_sds((b * m1, 16, 16)), (_STATS, 16)],
                    [(gr1, "row"), (nx1, "row"), (w1p, "full")], _QB)
    cnt1 = float(b * m1 * 16)
    h2, st2 = _call(functools.partial(_norm_lin_body, cnt=cnt1),
                    [_sds((b * m1, 16, 32)), (_STATS, 32)],
                    [(h1, "row"), (st1, "full"), (g1.reshape(1, -1), "full"),
                     (be1.reshape(1, -1), "full"), (w2, "full")], _QB)
    l1_feat = _call(functools.partial(_norm_max_body, cnt=cnt1),
                    [_sds((b * m1, 32))],
                    [(h2, "row"), (st2, "full"), (g2.reshape(1, -1), "full"),
                     (be2.reshape(1, -1), "full")], _QB)

    (w3, g3, be3), (w4, g4, be4) = params['sa2']
    t1 = jnp.concatenate([l1_xyz, l1_feat.reshape(b, m1, 32),
                          jnp.zeros((b, m1, 13), f32)], -1)
    gr2 = _sc_gather(t1.reshape(b * m1, 48), ball2.reshape(-1), 1024)
    gr2 = gr2.reshape(b * m2, 16, 48)
    nx2 = jnp.concatenate([l2_xyz, jnp.zeros((b, m2, 45), f32)], -1)
    nx2 = nx2.reshape(b * m2, 48)
    w3p = jnp.pad(w3, ((0, 0), (0, 13)))
    h3, st3 = _call(_grouped_lin_body,
                    [_sds((b * m2, 16, 32)), (_STATS, 32)],
                    [(gr2, "row"), (nx2, "row"), (w3p, "full")], _QB)
    cnt2 = float(b * m2 * 16)
    h4, st4 = _call(functools.partial(_norm_lin_body, cnt=cnt2),
                    [_sds((b * m2, 16, 64)), (_STATS, 64)],
                    [(h3, "row"), (st3, "full"), (g3.reshape(1, -1), "full"),
                     (be3.reshape(1, -1), "full"), (w4, "full")], _QB)
    l2_feat = _call(functools.partial(_norm_max_body, cnt=cnt2),
                    [_sds((b * m2, 64))],
                    [(h4, "row"), (st4, "full"), (g4.reshape(1, -1), "full"),
                     (be4.reshape(1, -1), "full")], _QB)

    (w5, g5, be5), = params['su1_mlp1']
    (w6, g6, be6), = params['su1_mlp2']
    t2 = jnp.concatenate([l2_feat.reshape(b, m2, 64), l2_xyz,
                          jnp.zeros((b, m2, 13), f32)], -1)
    gr3 = _sc_gather(t2.reshape(b * m2, 80), su.reshape(-1), 1024)
    gr3 = gr3.reshape(b * m1, 8, 80)
    p1 = jnp.concatenate([jnp.zeros((b, m1, 64), f32), l1_xyz,
                          jnp.zeros((b, m1, 13), f32)], -1)
    p1 = p1.reshape(b * m1, 80)
    w5p = jnp.pad(w5, ((0, 0), (0, 13)))
    h5, st5 = _call(_grouped_lin_body,
                    [_sds((b * m1, 8, 32)), (_STATS, 32)],
                    [(gr3, "row"), (p1, "row"), (w5p, "full")], _QB)
    cnt5 = float(b * m1 * 8)
    h6, st6 = _call(functools.partial(_su_b_body, cnt=cnt5),
                    [_sds((b * m1, 32)), (_STATS, 32)],
                    [(h5, "row"), (st5, "full"), (g5.reshape(1, -1), "full"),
                     (be5.reshape(1, -1), "full"), (l1_feat, "row"),
                     (w6, "full")], _QB)
    cnt6 = float(b * m1)
    t3 = _call(functools.partial(_su_c_body, cnt=cnt6),
               [_sds((b * m1, 48))],
               [(h6, "row"), (st6, "full"), (g6.reshape(1, -1), "full"),
                (be6.reshape(1, -1), "full"),
                (l1_xyz.reshape(b * m1, 3), "row")], _QB)

    (w7, g7, be7), = params['fp']
    gr4 = _sc_gather(t3, fp.reshape(-1), 1536)
    gr4 = gr4.reshape(b * n, 3, 48)
    h7, st7 = _call(_fp_a_body,
                    [_sds((b * n, 32)), (_STATS, 32)],
                    [(gr4, "row"), (pc.reshape(b * n, 3), "row"),
                     (fea.reshape(b * n, 10), "row"), (w7, "full")], _RB)
    cnt7 = float(b * n)
    y, sty = _call(functools.partial(_norm_relu_stats_body, cnt=cnt7),
                   [_sds((b * n, 32)), (_STATS, 32)],
                   [(h7, "row"), (st7, "full"), (g7.reshape(1, -1), "full"),
                    (be7.reshape(1, -1), "full")], _RB)
    gb1, bb1 = params['bn1']
    wc, cb = params['conv2']
    sf = _call(functools.partial(_final_body, cnt=cnt7),
               [_sds((b * n, 16))],
               [(y, "row"), (sty, "full"), (gb1.reshape(1, -1), "full"),
                (bb1.reshape(1, -1), "full"), (wc, "full"),
                (cb.reshape(1, -1), "full")], _RB)
    return jnp.concatenate([pc, sf.reshape(b, n, 16)], -1)


def kernel(points1, fea1, points2, fea2, params):
    n = points1.shape[1]
    pts = jnp.concatenate([points1, points2], 0)
    l1 = _fps(pts, n // 2)
    l2 = _fps(l1, n // 8)
    sf = _process(points1, fea1, params, l1[:2], l2[:2])
    tf = _process(points2, fea2, params, l1[2:], l2[2:])
    return (sf, tf)


# speedup vs baseline: 1.0094x; 1.0094x over previous
import functools

import jax
import jax.numpy as jnp
from jax import lax
from jax.experimental import pallas as pl
from jax.experimental.pallas import tpu as pltpu
from jax.experimental.pallas import tpu_sc as plsc

_R0 = 0.001
_QB = 256
_RB = 512


def _fps_body(rows_ref, xl_ref, xyz_ref, *, npoint, n, g):
    r = n // 128
    x = xl_ref[:, 0]
    y = xl_ref[:, 1]
    z = xl_ref[:, 2]
    lin = (lax.broadcasted_iota(jnp.int32, (g, r, 128), 1) * 128
           + lax.broadcasted_iota(jnp.int32, (g, r, 128), 2))
    d0 = jnp.full((g, r, 128), 1e10, dtype=jnp.float32)

    def body(i, carry):
        f, d = carry
        ccs = []
        for c in range(g):
            fc = f[c, 0, 0]
            cc = rows_ref[c, pl.ds(fc, 1), :]
            xyz_ref[c, pl.ds(i, 1), :] = cc
            ccs.append(cc[None])
        cc_all = jnp.concatenate(ccs, axis=0)
        cx = cc_all[:, :, 0:1]
        cy = cc_all[:, :, 1:2]
        cz = cc_all[:, :, 2:3]
        dx = x - cx
        dy = y - cy
        dz = z - cz
        dd = dx * dx + dy * dy + dz * dz
        d = jnp.minimum(d, dd)
        m = jnp.max(d, axis=(1, 2), keepdims=True)
        f2 = jnp.min(jnp.where(d == m, lin, n), axis=(1, 2), keepdims=True)
        return f2.astype(jnp.int32), d

    lax.fori_loop(0, npoint, body,
                  (jnp.zeros((g, 1, 1), jnp.int32), d0))


def _fps(xyz_rows, npoint):
    g, n, _ = xyz_rows.shape
    r = n // 128
    xl = xyz_rows.transpose(0, 2, 1).reshape(g, 3, r, 128)
    body = functools.partial(_fps_body, npoint=npoint, n=n, g=g)
    sxyz = pl.pallas_call(
        body,
        out_shape=jax.ShapeDtypeStruct((g, npoint, 3), jnp.float32),
    )(xyz_rows, xl)
    return sxyz


def _neighbor_body(q_ref, s_ref, o_ref, *, nsample, v, mode, r2):
    q = q_ref[0]
    s = s_ref[0]
    dg = lax.dot_general(q, s, (((1,), (0,)), ((), ())),
                         preferred_element_type=jnp.float32)
    qn = jnp.sum(q * q, axis=1, keepdims=True)
    sn = jnp.sum(s * s, axis=0, keepdims=True)
    d = -2.0 * dg
    d = d + qn
    d = d + sn
    qb = q.shape[0]
    lane = lax.broadcasted_iota(jnp.int32, (qb, v), 1)
    bidx = pl.program_id(0)
    if mode == "ball":
        g = jnp.where(d > r2, v, lane)
        cols = []
        for _ in range(nsample):
            m = jnp.min(g, axis=1, keepdims=True)
            cols.append(m)
            g = jnp.where(g == m, v, g)
        first = cols[0]
        cols = [jnp.where(c == v, first, c) for c in cols]
        res = jnp.minimum(jnp.concatenate(cols, axis=1), v - 1)
    else:
        cols = []
        for _ in range(nsample):
            m = jnp.min(d, axis=1, keepdims=True)
            j = jnp.min(jnp.where(d == m, lane, v), axis=1, keepdims=True)
            cols.append(j)
            d = jnp.where(lane == j, jnp.inf, d)
        res = jnp.concatenate(cols, axis=1)
    o_ref[0] = res + bidx * v


def _neighbor(queries, sources, nsample, mode, r2=0.0):
    b, m, _ = queries.shape
    v = sources.shape[1]
    qb = min(_QB, m)
    sl = sources.transpose(0, 2, 1)
    body = functools.partial(_neighbor_body, nsample=nsample, v=v,
                             mode=mode, r2=r2)
    idx = pl.pallas_call(
        body,
        grid=(b, m // qb),
        in_specs=[
            pl.BlockSpec((1, qb, 3), lambda i, j: (i, j, 0)),
            pl.BlockSpec((1, 3, v), lambda i, j: (i, 0, 0)),
        ],
        out_specs=pl.BlockSpec((1, qb, nsample), lambda i, j: (i, j, 0)),
        out_shape=jax.ShapeDtypeStruct((b, m, nsample), jnp.int32),
    )(queries, sl)
    return idx


def _sc_gather(table, idx, chunk):
    v, dd = table.shape
    (btot,) = idx.shape
    nc, ns = 2, 16
    nw = nc * ns
    b_per_w = btot // nw
    nchunks = b_per_w // chunk
    mesh = plsc.VectorSubcoreMesh(core_axis_name="c", subcore_axis_name="s")

    @functools.partial(
        pl.kernel,
        mesh=mesh,
        out_type=jax.ShapeDtypeStruct((btot, dd), jnp.float32),
        scratch_types=[
            pltpu.VMEM((chunk,), jnp.int32),
            pltpu.VMEM((chunk, dd), jnp.float32),
            pltpu.SemaphoreType.DMA,
        ],
    )
    def k(table_hbm, idx_hbm, out_hbm, idx_v, rows_v, sem):
        wid = lax.axis_index("s") * nc + lax.axis_index("c")
        base = wid * b_per_w
        for ci in range(nchunks):
            off = base + ci * chunk
            pltpu.sync_copy(idx_hbm.at[pl.ds(off, chunk)], idx_v)
            pltpu.async_copy(table_hbm.at[idx_v], rows_v, sem).wait()
            pltpu.sync_copy(rows_v, out_hbm.at[pl.ds(off, chunk)])

    return k(table, idx)


def _acc_stats(st_ref, y):
    s1 = jnp.sum(y, axis=0, keepdims=True)
    s2 = jnp.sum(y * y, axis=0, keepdims=True)

    @pl.when(pl.program_id(0) == 0)
    def _():
        st_ref[...] = jnp.zeros_like(st_ref)

    st_ref[...] += jnp.concatenate([s1, s2], axis=0)


def _bn_relu(h, stats, gamma, beta, cnt, relu=True):
    shp = (1,) * (h.ndim - 1) + (h.shape[-1],)
    mean = (stats[0:1, :] / cnt).reshape(shp)
    ex2 = (stats[1:2, :] / cnt).reshape(shp)
    var = ex2 - mean * mean
    y = gamma.reshape(shp) * (h - mean) / jnp.sqrt(var + 1e-5) + beta.reshape(shp)
    return jnp.maximum(y, 0.0) if relu else y


def _rowsT(h, w):
    return lax.dot_general(h, w, (((1,), (1,)), ((), ())),
                           preferred_element_type=jnp.float32)


def _grouped_lin_body(g_ref, nx_ref, w_ref, o_ref, st_ref):
    g = g_ref[...]
    nx = nx_ref[...]
    rb, k, dd = g.shape
    h = (g - nx[:, None, :]).reshape(rb * k, dd)
    y = _rowsT(h, w_ref[...])
    o_ref[...] = y.reshape(rb, k, y.shape[1])
    _acc_stats(st_ref, y)


def _norm_lin_body(h_ref, sti_ref, gam_ref, bet_ref, w_ref, o_ref, st_ref, *, cnt):
    h = h_ref[...]
    rb, k, cin = h.shape
    x = _bn_relu(h, sti_ref[...], gam_ref[...], bet_ref[...], cnt)
    y = _rowsT(x.reshape(rb * k, cin), w_ref[...])
    o_ref[...] = y.reshape(rb, k, y.shape[1])
    _acc_stats(st_ref, y)


def _norm_max_body(h_ref, sti_ref, gam_ref, bet_ref, o_ref, *, cnt):
    x = _bn_relu(h_ref[...], sti_ref[...], gam_ref[...], bet_ref[...], cnt)
    o_ref[...] = jnp.max(x, axis=1)


def _su_b_body(h_ref, sti_ref, gam_ref, bet_ref, f1_ref, w_ref, o_ref, st_ref, *, cnt):
    x = _bn_relu(h_ref[...], sti_ref[...], gam_ref[...], bet_ref[...], cnt)
    xm = jnp.max(x, axis=1)
    hc = jnp.concatenate([xm, f1_ref[...]], axis=1)
    y = _rowsT(hc, w_ref[...])
    o_ref[...] = y
    _acc_stats(st_ref, y)


def _su_c_body(h_ref, sti_ref, gam_ref, bet_ref, xyz_ref, o_ref, *, cnt):
    x = _bn_relu(h_ref[...], sti_ref[...], gam_ref[...], bet_ref[...], cnt)
    rb = x.shape[0]
    o_ref[...] = jnp.concatenate(
        [x, xyz_ref[...], jnp.zeros((rb, 93), jnp.float32)], axis=1)


def _fp_a_body(g_ref, pc_ref, fea_ref, w_ref, o_ref, st_ref):
    g = g_ref[...]
    p = pc_ref[...]
    f = fea_ref[...]
    px, py, pz = p[:, 0:1], p[:, 1:2], p[:, 2:3]
    ds = []
    for j in range(3):
        dx = g[:, j, 32:33] - px
        dy = g[:, j, 33:34] - py
        dz = g[:, j, 34:35] - pz
        ds.append(dx * dx + dy * dy + dz * dz)
    rc = [1.0 / (dj + 1e-8) for dj in ds]
    norm = rc[0] + rc[1] + rc[2]
    ws = [r / norm for r in rc]
    interp = (g[:, 0, 0:32] * ws[0] + g[:, 1, 0:32] * ws[1]
              + g[:, 2, 0:32] * ws[2])
    h = jnp.concatenate([interp, f], axis=1)
    y = _rowsT(h, w_ref[...])
    o_ref[...] = y
    _acc_stats(st_ref, y)


def _norm_relu_stats_body(h_ref, sti_ref, gam_ref, bet_ref, o_ref, st_ref, *, cnt):
    x = _bn_relu(h_ref[...], sti_ref[...], gam_ref[...], bet_ref[...], cnt)
    o_ref[...] = x
    _acc_stats(st_ref, x)


def _final_body(h_ref, sti_ref, gam_ref, bet_ref, w_ref, cb_ref, o_ref, *, cnt):
    x = _bn_relu(h_ref[...], sti_ref[...], gam_ref[...], bet_ref[...], cnt)
    o_ref[...] = _rowsT(x, w_ref[...]) + cb_ref[...]


def _full(shape):
    nd = len(shape)

    def im(i):
        return (0,) * nd

    return pl.BlockSpec(shape, im)


def _rowspec(blk_shape):
    nd = len(blk_shape)

    def im(i):
        return (i,) + (0,) * (nd - 1)

    return pl.BlockSpec(blk_shape, im)


_STATS = object()


def _call(body, outs, ins, rb):
    nrows = ins[0][0].shape[0]
    grid = (nrows // rb,)
    in_specs, args = [], []
    for a, kind in ins:
        if kind == "row":
            in_specs.append(_rowspec((rb,) + a.shape[1:]))
        else:
            in_specs.append(_full(a.shape))
        args.append(a)
    res_specs = []
    res_shapes = []
    for o in outs:
        if isinstance(o, tuple) and o[0] is _STATS:
            c = o[1]
            res_specs.append(_full((2, c)))
            res_shapes.append(jax.ShapeDtypeStruct((2, c), jnp.float32))
        else:
            res_specs.append(_rowspec((rb,) + o.shape[1:]))
            res_shapes.append(o)
    return pl.pallas_call(
        body,
        grid=grid,
        in_specs=in_specs,
        out_specs=res_specs if len(res_specs) > 1 else res_specs[0],
        out_shape=res_shapes if len(res_shapes) > 1 else res_shapes[0],
    )(*args)


def _sds(shape):
    return jax.ShapeDtypeStruct(shape, jnp.float32)


def _process(pc, fea, params, l1_xyz, l2_xyz):
    b, n, _ = pc.shape
    m1, m2 = n // 2, n // 8
    f32 = jnp.float32

    ball1 = _neighbor(l1_xyz, pc, 16, "ball", (_R0 * 8) ** 2)
    ball2 = _neighbor(l2_xyz, l1_xyz, 16, "ball", (_R0 * 16) ** 2)
    su = _neighbor(l1_xyz, l2_xyz, 8, "ball", (_R0 * 12) ** 2)
    fp = _neighbor(pc, l1_xyz, 3, "knn")

    (w1, g1, be1), (w2, g2, be2) = params['sa1']
    t0 = jnp.concatenate([pc, fea, jnp.zeros((b, n, 115), f32)], -1)
    gr1 = _sc_gather(t0.reshape(b * n, 128), ball1.reshape(-1), 512)
    gr1 = gr1.reshape(b * m1, 16, 128)
    nx1 = jnp.concatenate([l1_xyz, jnp.zeros((b, m1, 125), f32)], -1)
    nx1 = nx1.reshape(b * m1, 128)
    w1p = jnp.pad(w1, ((0, 0), (0, 115)))
    h1, st1 = _call(_grouped_lin_body,
                    [_sds((b * m1, 16, 16)), (_STATS, 16)],
                    [(gr1, "row"), (nx1, "row"), (w1p, "full")], _QB)
    cnt1 = float(b * m1 * 16)
    h2, st2 = _call(functools.partial(_norm_lin_body, cnt=cnt1),
                    [_sds((b * m1, 16, 32)), (_STATS, 32)],
                    [(h1, "row"), (st1, "full"), (g1.reshape(1, -1), "full"),
                     (be1.reshape(1, -1), "full"), (w2, "full")], _QB)
    l1_feat = _call(functools.partial(_norm_max_body, cnt=cnt1),
                    [_sds((b * m1, 32))],
                    [(h2, "row"), (st2, "full"), (g2.reshape(1, -1), "full"),
                     (be2.reshape(1, -1), "full")], _QB)

    (w3, g3, be3), (w4, g4, be4) = params['sa2']
    t1 = jnp.concatenate([l1_xyz, l1_feat.reshape(b, m1, 32),
                          jnp.zeros((b, m1, 93), f32)], -1)
    gr2 = _sc_gather(t1.reshape(b * m1, 128), ball2.reshape(-1), 512)
    gr2 = gr2.reshape(b * m2, 16, 128)
    nx2 = jnp.concatenate([l2_xyz, jnp.zeros((b, m2, 125), f32)], -1)
    nx2 = nx2.reshape(b * m2, 128)
    w3p = jnp.pad(w3, ((0, 0), (0, 93)))
    h3, st3 = _call(_grouped_lin_body,
                    [_sds((b * m2, 16, 32)), (_STATS, 32)],
                    [(gr2, "row"), (nx2, "row"), (w3p, "full")], _QB)
    cnt2 = float(b * m2 * 16)
    h4, st4 = _call(functools.partial(_norm_lin_body, cnt=cnt2),
                    [_sds((b * m2, 16, 64)), (_STATS, 64)],
                    [(h3, "row"), (st3, "full"), (g3.reshape(1, -1), "full"),
                     (be3.reshape(1, -1), "full"), (w4, "full")], _QB)
    l2_feat = _call(functools.partial(_norm_max_body, cnt=cnt2),
                    [_sds((b * m2, 64))],
                    [(h4, "row"), (st4, "full"), (g4.reshape(1, -1), "full"),
                     (be4.reshape(1, -1), "full")], _QB)

    (w5, g5, be5), = params['su1_mlp1']
    (w6, g6, be6), = params['su1_mlp2']
    t2 = jnp.concatenate([l2_feat.reshape(b, m2, 64), l2_xyz,
                          jnp.zeros((b, m2, 61), f32)], -1)
    gr3 = _sc_gather(t2.reshape(b * m2, 128), su.reshape(-1), 512)
    gr3 = gr3.reshape(b * m1, 8, 128)
    p1 = jnp.concatenate([jnp.zeros((b, m1, 64), f32), l1_xyz,
                          jnp.zeros((b, m1, 61), f32)], -1)
    p1 = p1.reshape(b * m1, 128)
    w5p = jnp.pad(w5, ((0, 0), (0, 61)))
    h5, st5 = _call(_grouped_lin_body,
                    [_sds((b * m1, 8, 32)), (_STATS, 32)],
                    [(gr3, "row"), (p1, "row"), (w5p, "full")], _QB)
    cnt5 = float(b * m1 * 8)
    h6, st6 = _call(functools.partial(_su_b_body, cnt=cnt5),
                    [_sds((b * m1, 32)), (_STATS, 32)],
                    [(h5, "row"), (st5, "full"), (g5.reshape(1, -1), "full"),
                     (be5.reshape(1, -1), "full"), (l1_feat, "row"),
                     (w6, "full")], _QB)
    cnt6 = float(b * m1)
    t3 = _call(functools.partial(_su_c_body, cnt=cnt6),
               [_sds((b * m1, 128))],
               [(h6, "row"), (st6, "full"), (g6.reshape(1, -1), "full"),
                (be6.reshape(1, -1), "full"),
                (l1_xyz.reshape(b * m1, 3), "row")], _QB)

    (w7, g7, be7), = params['fp']
    gr4 = _sc_gather(t3, fp.reshape(-1), 512)
    gr4 = gr4.reshape(b * n, 3, 128)
    h7, st7 = _call(_fp_a_body,
                    [_sds((b * n, 32)), (_STATS, 32)],
                    [(gr4, "row"), (pc.reshape(b * n, 3), "row"),
                     (fea.reshape(b * n, 10), "row"), (w7, "full")], _RB)
    cnt7 = float(b * n)
    y, sty = _call(functools.partial(_norm_relu_stats_body, cnt=cnt7),
                   [_sds((b * n, 32)), (_STATS, 32)],
                   [(h7, "row"), (st7, "full"), (g7.reshape(1, -1), "full"),
                    (be7.reshape(1, -1), "full")], _RB)
    gb1, bb1 = params['bn1']
    wc, cb = params['conv2']
    sf = _call(functools.partial(_final_body, cnt=cnt7),
               [_sds((b * n, 16))],
               [(y, "row"), (sty, "full"), (gb1.reshape(1, -1), "full"),
                (bb1.reshape(1, -1), "full"), (wc, "full"),
                (cb.reshape(1, -1), "full")], _RB)
    return jnp.concatenate([pc, sf.reshape(b, n, 16)], -1)


def kernel(points1, fea1, points2, fea2, params):
    n = points1.shape[1]
    pts = jnp.concatenate([points1, points2], 0)
    l1 = _fps(pts, n // 2)
    l2 = _fps(l1, n // 8)
    sf = _process(points1, fea1, params, l1[:2], l2[:2])
    tf = _process(points2, fea2, params, l1[2:], l2[2:])
    return (sf, tf)


# speedup vs baseline: 1.0370x; 1.0273x over previous
import functools

import jax
import jax.numpy as jnp
from jax import lax
from jax.experimental import pallas as pl
from jax.experimental.pallas import tpu as pltpu
from jax.experimental.pallas import tpu_sc as plsc

_R0 = 0.001
_QB = 256
_RB = 512


def _fps_body(xl_ref, xyz_ref, *, npoint, n, g):
    r = n // 128
    x = xl_ref[:, 0]
    y = xl_ref[:, 1]
    z = xl_ref[:, 2]
    lin = (lax.broadcasted_iota(jnp.int32, (g, r, 128), 1) * 128
           + lax.broadcasted_iota(jnp.int32, (g, r, 128), 2))
    d0 = jnp.full((g, r, 128), 1e10, dtype=jnp.float32)

    def body(i, carry):
        f, d = carry
        msk = lin == f
        zero = jnp.zeros((g, r, 128), jnp.float32)
        cx = jnp.sum(jnp.where(msk, x, zero), axis=(1, 2), keepdims=True)
        cy = jnp.sum(jnp.where(msk, y, zero), axis=(1, 2), keepdims=True)
        cz = jnp.sum(jnp.where(msk, z, zero), axis=(1, 2), keepdims=True)
        for c in range(g):
            cc = jnp.concatenate(
                [cx[c, :, 0:1], cy[c, :, 0:1], cz[c, :, 0:1]], axis=1)
            xyz_ref[c, pl.ds(i, 1), :] = cc
        dx = x - cx
        dy = y - cy
        dz = z - cz
        dd = dx * dx + dy * dy + dz * dz
        d = jnp.minimum(d, dd)
        m = jnp.max(d, axis=(1, 2), keepdims=True)
        f2 = jnp.min(jnp.where(d == m, lin, n), axis=(1, 2), keepdims=True)
        return f2.astype(jnp.int32), d

    lax.fori_loop(0, npoint, body,
                  (jnp.zeros((g, 1, 1), jnp.int32), d0))


def _fps(xyz_rows, npoint):
    g, n, _ = xyz_rows.shape
    r = n // 128
    xl = xyz_rows.transpose(0, 2, 1).reshape(g, 3, r, 128)
    body = functools.partial(_fps_body, npoint=npoint, n=n, g=g)
    sxyz = pl.pallas_call(
        body,
        out_shape=jax.ShapeDtypeStruct((g, npoint, 3), jnp.float32),
    )(xl)
    return sxyz


def _neighbor_body(q_ref, s_ref, o_ref, *, nsample, v, mode, r2):
    q = q_ref[0]
    s = s_ref[0]
    dg = lax.dot_general(q, s, (((1,), (0,)), ((), ())),
                         preferred_element_type=jnp.float32)
    qn = jnp.sum(q * q, axis=1, keepdims=True)
    sn = jnp.sum(s * s, axis=0, keepdims=True)
    d = -2.0 * dg
    d = d + qn
    d = d + sn
    qb = q.shape[0]
    lane = lax.broadcasted_iota(jnp.int32, (qb, v), 1)
    bidx = pl.program_id(0)
    if mode == "ball":
        g = jnp.where(d > r2, v, lane)
        cols = []
        for _ in range(nsample):
            m = jnp.min(g, axis=1, keepdims=True)
            cols.append(m)
            g = jnp.where(g == m, v, g)
        first = cols[0]
        cols = [jnp.where(c == v, first, c) for c in cols]
        res = jnp.minimum(jnp.concatenate(cols, axis=1), v - 1)
    else:
        cols = []
        for _ in range(nsample):
            m = jnp.min(d, axis=1, keepdims=True)
            j = jnp.min(jnp.where(d == m, lane, v), axis=1, keepdims=True)
            cols.append(j)
            d = jnp.where(lane == j, jnp.inf, d)
        res = jnp.concatenate(cols, axis=1)
    o_ref[0] = res + bidx * v


def _neighbor(queries, sources, nsample, mode, r2=0.0):
    b, m, _ = queries.shape
    v = sources.shape[1]
    qb = min(_QB, m)
    sl = sources.transpose(0, 2, 1)
    body = functools.partial(_neighbor_body, nsample=nsample, v=v,
                             mode=mode, r2=r2)
    idx = pl.pallas_call(
        body,
        compiler_params=pltpu.CompilerParams(
            dimension_semantics=("parallel", "parallel")),
        grid=(b, m // qb),
        in_specs=[
            pl.BlockSpec((1, qb, 3), lambda i, j: (i, j, 0)),
            pl.BlockSpec((1, 3, v), lambda i, j: (i, 0, 0)),
        ],
        out_specs=pl.BlockSpec((1, qb, nsample), lambda i, j: (i, j, 0)),
        out_shape=jax.ShapeDtypeStruct((b, m, nsample), jnp.int32),
    )(queries, sl)
    return idx


def _sc_gather(table, idx, chunk):
    v, dd = table.shape
    (btot,) = idx.shape
    nc, ns = 2, 16
    nw = nc * ns
    b_per_w = btot // nw
    nchunks = b_per_w // chunk
    mesh = plsc.VectorSubcoreMesh(core_axis_name="c", subcore_axis_name="s")

    @functools.partial(
        pl.kernel,
        mesh=mesh,
        out_type=jax.ShapeDtypeStruct((btot, dd), jnp.float32),
        scratch_types=[
            pltpu.VMEM((chunk,), jnp.int32),
            pltpu.VMEM((chunk, dd), jnp.float32),
            pltpu.SemaphoreType.DMA,
        ],
    )
    def k(table_hbm, idx_hbm, out_hbm, idx_v, rows_v, sem):
        wid = lax.axis_index("s") * nc + lax.axis_index("c")
        base = wid * b_per_w
        for ci in range(nchunks):
            off = base + ci * chunk
            pltpu.sync_copy(idx_hbm.at[pl.ds(off, chunk)], idx_v)
            pltpu.async_copy(table_hbm.at[idx_v], rows_v, sem).wait()
            pltpu.sync_copy(rows_v, out_hbm.at[pl.ds(off, chunk)])

    return k(table, idx)


def _acc_stats(st_ref, y):
    s1 = jnp.sum(y, axis=0, keepdims=True)
    s2 = jnp.sum(y * y, axis=0, keepdims=True)

    @pl.when(pl.program_id(0) == 0)
    def _():
        st_ref[...] = jnp.zeros_like(st_ref)

    st_ref[...] += jnp.concatenate([s1, s2], axis=0)


def _bn_relu(h, stats, gamma, beta, cnt, relu=True):
    shp = (1,) * (h.ndim - 1) + (h.shape[-1],)
    mean = (stats[0:1, :] / cnt).reshape(shp)
    ex2 = (stats[1:2, :] / cnt).reshape(shp)
    var = ex2 - mean * mean
    y = gamma.reshape(shp) * (h - mean) / jnp.sqrt(var + 1e-5) + beta.reshape(shp)
    return jnp.maximum(y, 0.0) if relu else y


def _rowsT(h, w):
    return lax.dot_general(h, w, (((1,), (1,)), ((), ())),
                           preferred_element_type=jnp.float32)


def _grouped_lin_body(g_ref, nx_ref, w_ref, o_ref, st_ref):
    g = g_ref[...]
    nx = nx_ref[...]
    rb, k, dd = g.shape
    h = (g - nx[:, None, :]).reshape(rb * k, dd)
    y = _rowsT(h, w_ref[...])
    o_ref[...] = y.reshape(rb, k, y.shape[1])
    _acc_stats(st_ref, y)


def _norm_lin_body(h_ref, sti_ref, gam_ref, bet_ref, w_ref, o_ref, st_ref, *, cnt):
    h = h_ref[...]
    rb, k, cin = h.shape
    x = _bn_relu(h, sti_ref[...], gam_ref[...], bet_ref[...], cnt)
    y = _rowsT(x.reshape(rb * k, cin), w_ref[...])
    o_ref[...] = y.reshape(rb, k, y.shape[1])
    _acc_stats(st_ref, y)


def _norm_max_body(h_ref, sti_ref, gam_ref, bet_ref, o_ref, *, cnt):
    x = _bn_relu(h_ref[...], sti_ref[...], gam_ref[...], bet_ref[...], cnt)
    o_ref[...] = jnp.max(x, axis=1)


def _su_b_body(h_ref, sti_ref, gam_ref, bet_ref, f1_ref, w_ref, o_ref, st_ref, *, cnt):
    x = _bn_relu(h_ref[...], sti_ref[...], gam_ref[...], bet_ref[...], cnt)
    xm = jnp.max(x, axis=1)
    hc = jnp.concatenate([xm, f1_ref[...]], axis=1)
    y = _rowsT(hc, w_ref[...])
    o_ref[...] = y
    _acc_stats(st_ref, y)


def _su_c_body(h_ref, sti_ref, gam_ref, bet_ref, xyz_ref, o_ref, *, cnt):
    x = _bn_relu(h_ref[...], sti_ref[...], gam_ref[...], bet_ref[...], cnt)
    rb = x.shape[0]
    o_ref[...] = jnp.concatenate(
        [x, xyz_ref[...], jnp.zeros((rb, 93), jnp.float32)], axis=1)


def _fp_a_body(g_ref, pc_ref, fea_ref, w_ref, o_ref, st_ref):
    g = g_ref[...]
    p = pc_ref[...]
    f = fea_ref[...]
    px, py, pz = p[:, 0:1], p[:, 1:2], p[:, 2:3]
    ds = []
    for j in range(3):
        dx = g[:, j, 32:33] - px
        dy = g[:, j, 33:34] - py
        dz = g[:, j, 34:35] - pz
        ds.append(dx * dx + dy * dy + dz * dz)
    rc = [1.0 / (dj + 1e-8) for dj in ds]
    norm = rc[0] + rc[1] + rc[2]
    ws = [r / norm for r in rc]
    interp = (g[:, 0, 0:32] * ws[0] + g[:, 1, 0:32] * ws[1]
              + g[:, 2, 0:32] * ws[2])
    h = jnp.concatenate([interp, f], axis=1)
    y = _rowsT(h, w_ref[...])
    o_ref[...] = y
    _acc_stats(st_ref, y)


def _norm_relu_stats_body(h_ref, sti_ref, gam_ref, bet_ref, o_ref, st_ref, *, cnt):
    x = _bn_relu(h_ref[...], sti_ref[...], gam_ref[...], bet_ref[...], cnt)
    o_ref[...] = x
    _acc_stats(st_ref, x)


def _final_body(h_ref, sti_ref, gam_ref, bet_ref, w_ref, cb_ref, o_ref, *, cnt):
    x = _bn_relu(h_ref[...], sti_ref[...], gam_ref[...], bet_ref[...], cnt)
    o_ref[...] = _rowsT(x, w_ref[...]) + cb_ref[...]


def _full(shape):
    nd = len(shape)

    def im(i):
        return (0,) * nd

    return pl.BlockSpec(shape, im)


def _rowspec(blk_shape):
    nd = len(blk_shape)

    def im(i):
        return (i,) + (0,) * (nd - 1)

    return pl.BlockSpec(blk_shape, im)


_STATS = object()


def _call(body, outs, ins, rb):
    nrows = ins[0][0].shape[0]
    grid = (nrows // rb,)
    in_specs, args = [], []
    for a, kind in ins:
        if kind == "row":
            in_specs.append(_rowspec((rb,) + a.shape[1:]))
        else:
            in_specs.append(_full(a.shape))
        args.append(a)
    res_specs = []
    res_shapes = []
    for o in outs:
        if isinstance(o, tuple) and o[0] is _STATS:
            c = o[1]
            res_specs.append(_full((2, c)))
            res_shapes.append(jax.ShapeDtypeStruct((2, c), jnp.float32))
        else:
            res_specs.append(_rowspec((rb,) + o.shape[1:]))
            res_shapes.append(o)
    return pl.pallas_call(
        body,
        grid=grid,
        in_specs=in_specs,
        out_specs=res_specs if len(res_specs) > 1 else res_specs[0],
        out_shape=res_shapes if len(res_shapes) > 1 else res_shapes[0],
    )(*args)


def _sds(shape):
    return jax.ShapeDtypeStruct(shape, jnp.float32)


def _process(pc, fea, params, l1_xyz, l2_xyz):
    b, n, _ = pc.shape
    m1, m2 = n // 2, n // 8
    f32 = jnp.float32

    ball1 = _neighbor(l1_xyz, pc, 16, "ball", (_R0 * 8) ** 2)
    ball2 = _neighbor(l2_xyz, l1_xyz, 16, "ball", (_R0 * 16) ** 2)
    su = _neighbor(l1_xyz, l2_xyz, 8, "ball", (_R0 * 12) ** 2)
    fp = _neighbor(pc, l1_xyz, 3, "knn")

    (w1, g1, be1), (w2, g2, be2) = params['sa1']
    t0 = jnp.concatenate([pc, fea, jnp.zeros((b, n, 115), f32)], -1)
    gr1 = _sc_gather(t0.reshape(b * n, 128), ball1.reshape(-1), 512)
    gr1 = gr1.reshape(b * m1, 16, 128)
    nx1 = jnp.concatenate([l1_xyz, jnp.zeros((b, m1, 125), f32)], -1)
    nx1 = nx1.reshape(b * m1, 128)
    w1p = jnp.pad(w1, ((0, 0), (0, 115)))
    h1, st1 = _call(_grouped_lin_body,
                    [_sds((b * m1, 16, 16)), (_STATS, 16)],
                    [(gr1, "row"), (nx1, "row"), (w1p, "full")], _QB)
    cnt1 = float(b * m1 * 16)
    h2, st2 = _call(functools.partial(_norm_lin_body, cnt=cnt1),
                    [_sds((b * m1, 16, 32)), (_STATS, 32)],
                    [(h1, "row"), (st1, "full"), (g1.reshape(1, -1), "full"),
                     (be1.reshape(1, -1), "full"), (w2, "full")], _QB)
    l1_feat = _call(functools.partial(_norm_max_body, cnt=cnt1),
                    [_sds((b * m1, 32))],
                    [(h2, "row"), (st2, "full"), (g2.reshape(1, -1), "full"),
                     (be2.reshape(1, -1), "full")], _QB)

    (w3, g3, be3), (w4, g4, be4) = params['sa2']
    t1 = jnp.concatenate([l1_xyz, l1_feat.reshape(b, m1, 32),
                          jnp.zeros((b, m1, 93), f32)], -1)
    gr2 = _sc_gather(t1.reshape(b * m1, 128), ball2.reshape(-1), 512)
    gr2 = gr2.reshape(b * m2, 16, 128)
    nx2 = jnp.concatenate([l2_xyz, jnp.zeros((b, m2, 125), f32)], -1)
    nx2 = nx2.reshape(b * m2, 128)
    w3p = jnp.pad(w3, ((0, 0), (0, 93)))
    h3, st3 = _call(_grouped_lin_body,
                    [_sds((b * m2, 16, 32)), (_STATS, 32)],
                    [(gr2, "row"), (nx2, "row"), (w3p, "full")], _QB)
    cnt2 = float(b * m2 * 16)
    h4, st4 = _call(functools.partial(_norm_lin_body, cnt=cnt2),
                    [_sds((b * m2, 16, 64)), (_STATS, 64)],
                    [(h3, "row"), (st3, "full"), (g3.reshape(1, -1), "full"),
                     (be3.reshape(1, -1), "full"), (w4, "full")], _QB)
    l2_feat = _call(functools.partial(_norm_max_body, cnt=cnt2),
                    [_sds((b * m2, 64))],
                    [(h4, "row"), (st4, "full"), (g4.reshape(1, -1), "full"),
                     (be4.reshape(1, -1), "full")], _QB)

    (w5, g5, be5), = params['su1_mlp1']
    (w6, g6, be6), = params['su1_mlp2']
    t2 = jnp.concatenate([l2_feat.reshape(b, m2, 64), l2_xyz,
                          jnp.zeros((b, m2, 61), f32)], -1)
    gr3 = _sc_gather(t2.reshape(b * m2, 128), su.reshape(-1), 512)
    gr3 = gr3.reshape(b * m1, 8, 128)
    p1 = jnp.concatenate([jnp.zeros((b, m1, 64), f32), l1_xyz,
                          jnp.zeros((b, m1, 61), f32)], -1)
    p1 = p1.reshape(b * m1, 128)
    w5p = jnp.pad(w5, ((0, 0), (0, 61)))
    h5, st5 = _call(_grouped_lin_body,
                    [_sds((b * m1, 8, 32)), (_STATS, 32)],
                    [(gr3, "row"), (p1, "row"), (w5p, "full")], _QB)
    cnt5 = float(b * m1 * 8)
    h6, st6 = _call(functools.partial(_su_b_body, cnt=cnt5),
                    [_sds((b * m1, 32)), (_STATS, 32)],
                    [(h5, "row"), (st5, "full"), (g5.reshape(1, -1), "full"),
                     (be5.reshape(1, -1), "full"), (l1_feat, "row"),
                     (w6, "full")], _QB)
    cnt6 = float(b * m1)
    t3 = _call(functools.partial(_su_c_body, cnt=cnt6),
               [_sds((b * m1, 128))],
               [(h6, "row"), (st6, "full"), (g6.reshape(1, -1), "full"),
                (be6.reshape(1, -1), "full"),
                (l1_xyz.reshape(b * m1, 3), "row")], _QB)

    (w7, g7, be7), = params['fp']
    gr4 = _sc_gather(t3, fp.reshape(-1), 512)
    gr4 = gr4.reshape(b * n, 3, 128)
    h7, st7 = _call(_fp_a_body,
                    [_sds((b * n, 32)), (_STATS, 32)],
                    [(gr4, "row"), (pc.reshape(b * n, 3), "row"),
                     (fea.reshape(b * n, 10), "row"), (w7, "full")], _RB)
    cnt7 = float(b * n)
    y, sty = _call(functools.partial(_norm_relu_stats_body, cnt=cnt7),
                   [_sds((b * n, 32)), (_STATS, 32)],
                   [(h7, "row"), (st7, "full"), (g7.reshape(1, -1), "full"),
                    (be7.reshape(1, -1), "full")], _RB)
    gb1, bb1 = params['bn1']
    wc, cb = params['conv2']
    sf = _call(functools.partial(_final_body, cnt=cnt7),
               [_sds((b * n, 16))],
               [(y, "row"), (sty, "full"), (gb1.reshape(1, -1), "full"),
                (bb1.reshape(1, -1), "full"), (wc, "full"),
                (cb.reshape(1, -1), "full")], _RB)
    return jnp.concatenate([pc, sf.reshape(b, n, 16)], -1)


def kernel(points1, fea1, points2, fea2, params):
    n = points1.shape[1]
    pts = jnp.concatenate([points1, points2], 0)
    l1 = _fps(pts, n // 2)
    l2 = _fps(l1, n // 8)
    sf = _process(points1, fea1, params, l1[:2], l2[:2])
    tf = _process(points2, fea2, params, l1[2:], l2[2:])
    return (sf, tf)
